# baseline (speedup 1.0000x reference)
import functools

import jax
import jax.numpy as jnp
from jax import lax
from jax.experimental import pallas as pl
from jax.experimental.pallas import tpu as pltpu
from jax.experimental.pallas import tpu_sc as plsc

NC = 2
NS = 16
LANES = 16
CH = 400


def _sc_degree(dst3d, ew3d, zeros_n, n_nodes):
  rpw = dst3d.shape[1]
  mesh = plsc.VectorSubcoreMesh(core_axis_name="c", subcore_axis_name="s")

  @functools.partial(
      pl.kernel,
      out_type=jax.ShapeDtypeStruct((NC, 1, n_nodes), jnp.float32),
      mesh=mesh,
      compiler_params=pltpu.CompilerParams(use_tc_tiling_on_sc=False),
      scratch_types=[
          pltpu.VMEM((rpw, CH), jnp.int32),
          pltpu.VMEM((rpw, CH), jnp.float32),
          pltpu.VMEM_SHARED((n_nodes,), jnp.float32),
          pltpu.SemaphoreType.DMA,
      ],
  )
  def deg_kernel(dst_hbm, ew_hbm, z_hbm, out_hbm, dst_v, ew_v, acc, sem):
    cid = lax.axis_index("c")
    sid = lax.axis_index("s")
    wid = cid * NS + sid

    @pl.when(sid == 0)
    def _():
      pltpu.sync_copy(z_hbm, acc)

    pltpu.sync_copy(dst_hbm.at[wid], dst_v)
    pltpu.sync_copy(ew_hbm.at[wid], ew_v)
    plsc.subcore_barrier()

    @pl.loop(0, rpw)
    def _(j):
      pltpu.async_copy(ew_v.at[j], acc.at[dst_v.at[j]], sem, add=True).wait()

    plsc.subcore_barrier()

    @pl.when(sid == 0)
    def _():
      pltpu.sync_copy(acc, out_hbm.at[cid, 0])

  return deg_kernel(dst3d, ew3d, zeros_n)


def _sc_scatter(src3d, dst3d, ewt3d, g, zeros_nf, n_nodes, feat):
  rpw = src3d.shape[1]
  nslice = feat // LANES
  rps = (n_nodes // NS) // 8 * 8
  tail = n_nodes - rps * NS
  mesh = plsc.VectorSubcoreMesh(core_axis_name="c", subcore_axis_name="s")

  @functools.partial(
      pl.kernel,
      out_type=jax.ShapeDtypeStruct((NC, n_nodes, feat), jnp.float32),
      mesh=mesh,
      compiler_params=pltpu.CompilerParams(use_tc_tiling_on_sc=False),
      scratch_types=[
          pltpu.VMEM((rpw, CH), jnp.int32),
          pltpu.VMEM((rpw, CH), jnp.int32),
          pltpu.VMEM((CH, rpw), jnp.float32),
          pltpu.VMEM((CH, feat), jnp.float32),
          pltpu.VMEM_SHARED((n_nodes, feat), jnp.float32),
          pltpu.SemaphoreType.DMA,
          pltpu.SemaphoreType.DMA,
      ],
  )
  def scat_kernel(src_hbm, dst_hbm, ew_hbm, g_hbm, z_hbm, out_hbm,
                  src_v, dst_v, ew_v, rows_v, acc, gsem, ssem):
    cid = lax.axis_index("c")
    sid = lax.axis_index("s")
    wid = cid * NS + sid

    pltpu.sync_copy(z_hbm.at[pl.ds(sid * rps, rps)],
                    acc.at[pl.ds(sid * rps, rps)])
    if tail:
      @pl.when(sid == 0)
      def _():
        pltpu.sync_copy(z_hbm.at[pl.ds(rps * NS, tail)],
                        acc.at[pl.ds(rps * NS, tail)])
    pltpu.sync_copy(src_hbm.at[wid], src_v)
    pltpu.sync_copy(dst_hbm.at[wid], dst_v)
    pltpu.sync_copy(ew_hbm.at[wid], ew_v)
    plsc.subcore_barrier()

    @pl.loop(0, rpw)
    def _(j):
      pltpu.async_copy(g_hbm.at[src_v.at[j]], rows_v, gsem).wait()
      w_col = ew_v[:, pl.ds(j, 1)]
      rows_v[...] = rows_v[...] * w_col
      pltpu.async_copy(rows_v, acc.at[dst_v.at[j]], ssem, add=True).wait()

    plsc.subcore_barrier()
    pltpu.sync_copy(acc.at[pl.ds(sid * rps, rps)],
                    out_hbm.at[cid, pl.ds(sid * rps, rps)])
    if tail:
      @pl.when(sid == 0)
      def _():
        pltpu.sync_copy(acc.at[pl.ds(rps * NS, tail)],
                        out_hbm.at[cid, pl.ds(rps * NS, tail)])

  return scat_kernel(src3d, dst3d, ewt3d, g, zeros_nf)


def _tc_encode(x, sv, f1w, f1b, f2w, f2b, w1a_t, w1b, deg_part):
  n = x.shape[0]
  h1 = w1a_t.shape[1]

  def body(x_ref, sv_ref, f1w_ref, f1b_ref, f2w_ref, f2b_ref, wa_ref, wb_ref,
           dp_ref, g1_ref, dinv_ref):
    sh = jnp.dot(sv_ref[...], f1w_ref[...], preferred_element_type=jnp.float32)
    sh = jnp.maximum(sh + f1b_ref[...], 0.0)
    senc = jnp.dot(sh, f2w_ref[...], preferred_element_type=jnp.float32)
    senc = senc + f2b_ref[...]
    hh = jnp.dot(x_ref[...], wa_ref[...], preferred_element_type=jnp.float32)
    hh = hh + jnp.dot(senc, wb_ref[...], preferred_element_type=jnp.float32)
    deg = dp_ref[0] + dp_ref[1] + 1.0
    dinv = lax.rsqrt(deg)
    g1_ref[...] = hh * dinv
    dinv_ref[...] = dinv

  return pl.pallas_call(
      body,
      out_shape=[
          jax.ShapeDtypeStruct((n, h1), jnp.float32),
          jax.ShapeDtypeStruct((n, 1), jnp.float32),
      ],
  )(x, sv, f1w, f1b, f2w, f2b, w1a_t, w1b, deg_part)


def _tc_mid(acc1, g1, dinv, b1, w2_t):
  n, h1 = g1.shape
  h2 = w2_t.shape[1]

  def body(a_ref, g_ref, d_ref, b_ref, w_ref, g2_ref):
    x1 = d_ref[...] * (a_ref[0] + a_ref[1] + g_ref[...]) + b_ref[...]
    act = jnp.where(x1 >= 0, x1, 0.01 * x1)
    hh = jnp.dot(act, w_ref[...], preferred_element_type=jnp.float32)
    g2_ref[...] = hh * d_ref[...]

  return pl.pallas_call(
      body,
      out_shape=jax.ShapeDtypeStruct((n, h2), jnp.float32),
  )(acc1, g1, dinv, b1, w2_t)


def _tc_final(acc2, g2, dinv, b2, fc_t, fc_b):
  n = g2.shape[0]

  def body(a_ref, g_ref, d_ref, b_ref, w_ref, fb_ref, out_ref):
    x2 = d_ref[...] * (a_ref[0] + a_ref[1] + g_ref[...]) + b_ref[...]
    act = jnp.where(x2 >= 0, x2, 0.01 * x2)
    out_ref[...] = (
        jnp.dot(act, w_ref[...], preferred_element_type=jnp.float32)
        + fb_ref[...]
    )

  return pl.pallas_call(
      body,
      out_shape=jax.ShapeDtypeStruct((n, 1), jnp.float32),
  )(acc2, g2, dinv, b2, fc_t, fc_b)


def kernel(x, edge_index, edge_weight, s_values, s_fc1_w, s_fc1_b, s_fc2_w,
           s_fc2_b, conv1_w, conv1_b, conv2_w, conv2_b, fc1_w, fc1_b):
  n, d = x.shape
  e = edge_weight.shape[0]
  h1 = conv1_w.shape[0]
  h2 = conv2_w.shape[0]

  nw = NC * NS
  rpw = e // (CH * nw)
  src3d = edge_index[0].reshape(nw, rpw, CH)
  dst3d = edge_index[1].reshape(nw, rpw, CH)
  ew3d = edge_weight.reshape(nw, rpw, CH)
  ewt3d = ew3d.transpose(0, 2, 1)
  zeros_n = jnp.zeros((n,), jnp.float32)
  zeros_nf1 = jnp.zeros((n, h1), jnp.float32)
  zeros_nf2 = jnp.zeros((n, h2), jnp.float32)

  deg_part = _sc_degree(dst3d, ew3d, zeros_n, n)
  deg_part3 = deg_part.reshape(NC, n, 1)

  g1, dinv = _tc_encode(
      x,
      s_values.reshape(1, -1),
      s_fc1_w.T,
      s_fc1_b.reshape(1, -1),
      s_fc2_w.T,
      s_fc2_b.reshape(1, -1),
      conv1_w[:, :d].T,
      conv1_w[:, d:].T,
      deg_part3,
  )

  acc1 = _sc_scatter(src3d, dst3d, ewt3d, g1, zeros_nf1, n, h1)
  g2 = _tc_mid(acc1, g1, dinv, conv1_b.reshape(1, -1), conv2_w.T)
  acc2 = _sc_scatter(src3d, dst3d, ewt3d, g2, zeros_nf2, n, h2)
  out = _tc_final(acc2, g2, dinv, conv2_b.reshape(1, -1), fc1_w.T,
                  fc1_b.reshape(1, -1))
  return out


# speedup vs baseline: 1.0915x; 1.0915x over previous
import functools

import jax
import jax.numpy as jnp
from jax import lax
from jax.experimental import pallas as pl
from jax.experimental.pallas import tpu as pltpu
from jax.experimental.pallas import tpu_sc as plsc

NC = 2
NS = 16
LANES = 16
CH = 200


def _sc_degree(dst3d, ew3d, zeros_n, n_nodes):
  rpw = dst3d.shape[1]
  mesh = plsc.VectorSubcoreMesh(core_axis_name="c", subcore_axis_name="s")

  @functools.partial(
      pl.kernel,
      out_type=jax.ShapeDtypeStruct((NC, 1, n_nodes), jnp.float32),
      mesh=mesh,
      compiler_params=pltpu.CompilerParams(use_tc_tiling_on_sc=False),
      scratch_types=[
          pltpu.VMEM((rpw, CH), jnp.int32),
          pltpu.VMEM((rpw, CH), jnp.float32),
          pltpu.VMEM_SHARED((n_nodes,), jnp.float32),
          pltpu.SemaphoreType.DMA,
      ],
  )
  def deg_kernel(dst_hbm, ew_hbm, z_hbm, out_hbm, dst_v, ew_v, acc, sem):
    cid = lax.axis_index("c")
    sid = lax.axis_index("s")
    wid = cid * NS + sid

    @pl.when(sid == 0)
    def _():
      pltpu.sync_copy(z_hbm, acc)

    pltpu.sync_copy(dst_hbm.at[wid], dst_v)
    pltpu.sync_copy(ew_hbm.at[wid], ew_v)
    plsc.subcore_barrier()

    @pl.loop(0, rpw)
    def _(j):
      pltpu.async_copy(ew_v.at[j], acc.at[dst_v.at[j]], sem, add=True).wait()

    plsc.subcore_barrier()

    @pl.when(sid == 0)
    def _():
      pltpu.sync_copy(acc, out_hbm.at[cid, 0])

  return deg_kernel(dst3d, ew3d, zeros_n)


def _sc_scatter(src3d, dst3d, ewt3d, g, zeros_nf, n_nodes, feat):
  rpw = src3d.shape[1]
  nslice = feat // LANES
  rps = (n_nodes // NS) // 8 * 8
  tail = n_nodes - rps * NS
  mesh = plsc.VectorSubcoreMesh(core_axis_name="c", subcore_axis_name="s")

  @functools.partial(
      pl.kernel,
      out_type=jax.ShapeDtypeStruct((NC, n_nodes, feat), jnp.float32),
      mesh=mesh,
      compiler_params=pltpu.CompilerParams(use_tc_tiling_on_sc=False),
      scratch_types=[
          pltpu.VMEM((rpw, CH), jnp.int32),
          pltpu.VMEM((rpw, CH), jnp.int32),
          pltpu.VMEM((CH, rpw), jnp.float32),
          pltpu.VMEM((CH, feat), jnp.float32),
          pltpu.VMEM((CH, feat), jnp.float32),
          pltpu.VMEM_SHARED((n_nodes, feat), jnp.float32),
          pltpu.SemaphoreType.DMA,
          pltpu.SemaphoreType.DMA,
          pltpu.SemaphoreType.DMA,
          pltpu.SemaphoreType.DMA,
      ],
  )
  def scat_kernel(src_hbm, dst_hbm, ew_hbm, g_hbm, z_hbm, out_hbm,
                  src_v, dst_v, ew_v, rows0_v, rows1_v, acc,
                  g0sem, g1sem, s0sem, s1sem):
    cid = lax.axis_index("c")
    sid = lax.axis_index("s")
    wid = cid * NS + sid

    pltpu.sync_copy(z_hbm.at[pl.ds(sid * rps, rps)],
                    acc.at[pl.ds(sid * rps, rps)])
    if tail:
      @pl.when(sid == 0)
      def _():
        pltpu.sync_copy(z_hbm.at[pl.ds(rps * NS, tail)],
                        acc.at[pl.ds(rps * NS, tail)])
    pltpu.sync_copy(src_hbm.at[wid], src_v)
    pltpu.sync_copy(dst_hbm.at[wid], dst_v)
    pltpu.sync_copy(ew_hbm.at[wid], ew_v)
    plsc.subcore_barrier()

    bufs = ((rows0_v, g0sem, s0sem), (rows1_v, g1sem, s1sem))

    def g_start(j, b):
      rows, gsem, _ = bufs[b]
      pltpu.async_copy(g_hbm.at[src_v.at[j]], rows, gsem)

    def g_wait(b):
      rows, gsem, _ = bufs[b]
      pltpu.make_async_copy(g_hbm.at[src_v.at[0]], rows, gsem).wait()

    def scale(j, b):
      rows = bufs[b][0]
      rows[...] = rows[...] * ew_v[:, pl.ds(j, 1)]

    def s_start(j, b):
      rows, _, ssem = bufs[b]
      pltpu.async_copy(rows, acc.at[dst_v.at[j]], ssem, add=True)

    def s_wait(b):
      rows, _, ssem = bufs[b]
      pltpu.make_async_copy(rows, acc.at[dst_v.at[0]], ssem).wait()

    g_start(0, 0)
    g_start(1, 1)
    g_wait(0)
    scale(0, 0)
    s_start(0, 0)
    g_wait(1)
    scale(1, 1)
    s_wait(0)
    if rpw > 2:
      g_start(2, 0)
    s_start(1, 1)

    if rpw > 2:
      @pl.loop(2, rpw, step=2)
      def _(j):
        g_wait(0)
        scale(j, 0)
        s_wait(1)
        g_start(j + 1, 1)
        s_start(j, 0)
        g_wait(1)
        scale(j + 1, 1)
        s_wait(0)

        @pl.when(j + 2 < rpw)
        def _():
          g_start(j + 2, 0)

        s_start(j + 1, 1)

    s_wait(1)
    plsc.subcore_barrier()
    pltpu.sync_copy(acc.at[pl.ds(sid * rps, rps)],
                    out_hbm.at[cid, pl.ds(sid * rps, rps)])
    if tail:
      @pl.when(sid == 0)
      def _():
        pltpu.sync_copy(acc.at[pl.ds(rps * NS, tail)],
                        out_hbm.at[cid, pl.ds(rps * NS, tail)])

  return scat_kernel(src3d, dst3d, ewt3d, g, zeros_nf)


def _tc_encode(x, sv, f1w, f1b, f2w, f2b, w1a_t, w1b, deg_part):
  n = x.shape[0]
  h1 = w1a_t.shape[1]

  def body(x_ref, sv_ref, f1w_ref, f1b_ref, f2w_ref, f2b_ref, wa_ref, wb_ref,
           dp_ref, g1_ref, dinv_ref):
    sh = jnp.dot(sv_ref[...], f1w_ref[...], preferred_element_type=jnp.float32)
    sh = jnp.maximum(sh + f1b_ref[...], 0.0)
    senc = jnp.dot(sh, f2w_ref[...], preferred_element_type=jnp.float32)
    senc = senc + f2b_ref[...]
    hh = jnp.dot(x_ref[...], wa_ref[...], preferred_element_type=jnp.float32)
    hh = hh + jnp.dot(senc, wb_ref[...], preferred_element_type=jnp.float32)
    deg = dp_ref[0] + dp_ref[1] + 1.0
    dinv = lax.rsqrt(deg)
    g1_ref[...] = hh * dinv
    dinv_ref[...] = dinv

  return pl.pallas_call(
      body,
      out_shape=[
          jax.ShapeDtypeStruct((n, h1), jnp.float32),
          jax.ShapeDtypeStruct((n, 1), jnp.float32),
      ],
  )(x, sv, f1w, f1b, f2w, f2b, w1a_t, w1b, deg_part)


def _tc_mid(acc1, g1, dinv, b1, w2_t):
  n, h1 = g1.shape
  h2 = w2_t.shape[1]

  def body(a_ref, g_ref, d_ref, b_ref, w_ref, g2_ref):
    x1 = d_ref[...] * (a_ref[0] + a_ref[1] + g_ref[...]) + b_ref[...]
    act = jnp.where(x1 >= 0, x1, 0.01 * x1)
    hh = jnp.dot(act, w_ref[...], preferred_element_type=jnp.float32)
    g2_ref[...] = hh * d_ref[...]

  return pl.pallas_call(
      body,
      out_shape=jax.ShapeDtypeStruct((n, h2), jnp.float32),
  )(acc1, g1, dinv, b1, w2_t)


def _tc_final(acc2, g2, dinv, b2, fc_t, fc_b):
  n = g2.shape[0]

  def body(a_ref, g_ref, d_ref, b_ref, w_ref, fb_ref, out_ref):
    x2 = d_ref[...] * (a_ref[0] + a_ref[1] + g_ref[...]) + b_ref[...]
    act = jnp.where(x2 >= 0, x2, 0.01 * x2)
    out_ref[...] = (
        jnp.dot(act, w_ref[...], preferred_element_type=jnp.float32)
        + fb_ref[...]
    )

  return pl.pallas_call(
      body,
      out_shape=jax.ShapeDtypeStruct((n, 1), jnp.float32),
  )(acc2, g2, dinv, b2, fc_t, fc_b)


def kernel(x, edge_index, edge_weight, s_values, s_fc1_w, s_fc1_b, s_fc2_w,
           s_fc2_b, conv1_w, conv1_b, conv2_w, conv2_b, fc1_w, fc1_b):
  n, d = x.shape
  e = edge_weight.shape[0]
  h1 = conv1_w.shape[0]
  h2 = conv2_w.shape[0]

  nw = NC * NS
  rpw = e // (CH * nw)
  src3d = edge_index[0].reshape(nw, rpw, CH)
  dst3d = edge_index[1].reshape(nw, rpw, CH)
  ew3d = edge_weight.reshape(nw, rpw, CH)
  ewt3d = ew3d.transpose(0, 2, 1)
  zeros_n = jnp.zeros((n,), jnp.float32)
  zeros_nf1 = jnp.zeros((n, h1), jnp.float32)
  zeros_nf2 = jnp.zeros((n, h2), jnp.float32)

  deg_part = _sc_degree(dst3d, ew3d, zeros_n, n)
  deg_part3 = deg_part.reshape(NC, n, 1)

  g1, dinv = _tc_encode(
      x,
      s_values.reshape(1, -1),
      s_fc1_w.T,
      s_fc1_b.reshape(1, -1),
      s_fc2_w.T,
      s_fc2_b.reshape(1, -1),
      conv1_w[:, :d].T,
      conv1_w[:, d:].T,
      deg_part3,
  )

  acc1 = _sc_scatter(src3d, dst3d, ewt3d, g1, zeros_nf1, n, h1)
  g2 = _tc_mid(acc1, g1, dinv, conv1_b.reshape(1, -1), conv2_w.T)
  acc2 = _sc_scatter(src3d, dst3d, ewt3d, g2, zeros_nf2, n, h2)
  out = _tc_final(acc2, g2, dinv, conv2_b.reshape(1, -1), fc1_w.T,
                  fc1_b.reshape(1, -1))
  return out


# speedup vs baseline: 1.2681x; 1.1618x over previous
import functools

import jax
import jax.numpy as jnp
from jax import lax
from jax.experimental import pallas as pl
from jax.experimental.pallas import tpu as pltpu
from jax.experimental.pallas import tpu_sc as plsc

NC = 2
NS = 16
LANES = 16
CH = 200


def _sc_degree(dst3d, ew3d, zeros_n, n_nodes):
  rpw = dst3d.shape[1]
  mesh = plsc.VectorSubcoreMesh(core_axis_name="c", subcore_axis_name="s")

  @functools.partial(
      pl.kernel,
      out_type=jax.ShapeDtypeStruct((NC, 1, n_nodes), jnp.float32),
      mesh=mesh,
      compiler_params=pltpu.CompilerParams(use_tc_tiling_on_sc=False),
      scratch_types=[
          pltpu.VMEM((rpw, CH), jnp.int32),
          pltpu.VMEM((rpw, CH), jnp.float32),
          pltpu.VMEM_SHARED((n_nodes,), jnp.float32),
          pltpu.SemaphoreType.DMA,
      ],
  )
  def deg_kernel(dst_hbm, ew_hbm, z_hbm, out_hbm, dst_v, ew_v, acc, sem):
    cid = lax.axis_index("c")
    sid = lax.axis_index("s")
    wid = cid * NS + sid

    @pl.when(sid == 0)
    def _():
      pltpu.sync_copy(z_hbm, acc)

    pltpu.sync_copy(dst_hbm.at[wid], dst_v)
    pltpu.sync_copy(ew_hbm.at[wid], ew_v)
    plsc.subcore_barrier()

    @pl.loop(0, rpw)
    def _(j):
      pltpu.async_copy(ew_v.at[j], acc.at[dst_v.at[j]], sem, add=True).wait()

    plsc.subcore_barrier()

    @pl.when(sid == 0)
    def _():
      pltpu.sync_copy(acc, out_hbm.at[cid, 0])

  return deg_kernel(dst3d, ew3d, zeros_n)


def _sc_scatter(src3d, dst3d, ewt3d, g, zeros_nf, n_nodes, feat):
  rpw = src3d.shape[1]
  nslice = feat // LANES
  rps = (n_nodes // NS) // 8 * 8
  tail = n_nodes - rps * NS
  mesh = plsc.VectorSubcoreMesh(core_axis_name="c", subcore_axis_name="s")

  @functools.partial(
      pl.kernel,
      out_type=jax.ShapeDtypeStruct((NC, n_nodes, feat), jnp.float32),
      mesh=mesh,
      compiler_params=pltpu.CompilerParams(use_tc_tiling_on_sc=False),
      scratch_types=[
          pltpu.VMEM((rpw, CH), jnp.int32),
          pltpu.VMEM((rpw, CH), jnp.int32),
          pltpu.VMEM((CH, rpw), jnp.float32),
          pltpu.VMEM((CH, feat), jnp.float32),
          pltpu.VMEM((CH, feat), jnp.float32),
          pltpu.VMEM_SHARED((n_nodes, feat), jnp.float32),
          pltpu.VMEM_SHARED((n_nodes, feat), jnp.float32),
          pltpu.SemaphoreType.DMA,
          pltpu.SemaphoreType.DMA,
          pltpu.SemaphoreType.DMA,
          pltpu.SemaphoreType.DMA,
      ],
  )
  def scat_kernel(src_hbm, dst_hbm, ew_hbm, g_hbm, z_hbm, out_hbm,
                  src_v, dst_v, ew_v, rows0_v, rows1_v, acc, g_s,
                  g0sem, g1sem, s0sem, s1sem):
    cid = lax.axis_index("c")
    sid = lax.axis_index("s")
    wid = cid * NS + sid

    pltpu.sync_copy(z_hbm.at[pl.ds(sid * rps, rps)],
                    acc.at[pl.ds(sid * rps, rps)])
    pltpu.sync_copy(g_hbm.at[pl.ds(sid * rps, rps)],
                    g_s.at[pl.ds(sid * rps, rps)])
    if tail:
      @pl.when(sid == 0)
      def _():
        pltpu.sync_copy(z_hbm.at[pl.ds(rps * NS, tail)],
                        acc.at[pl.ds(rps * NS, tail)])
        pltpu.sync_copy(g_hbm.at[pl.ds(rps * NS, tail)],
                        g_s.at[pl.ds(rps * NS, tail)])
    pltpu.sync_copy(src_hbm.at[wid], src_v)
    pltpu.sync_copy(dst_hbm.at[wid], dst_v)
    pltpu.sync_copy(ew_hbm.at[wid], ew_v)
    plsc.subcore_barrier()

    bufs = ((rows0_v, g0sem, s0sem), (rows1_v, g1sem, s1sem))

    def g_start(j, b):
      rows, gsem, _ = bufs[b]
      pltpu.async_copy(g_s.at[src_v.at[j]], rows, gsem)

    def g_wait(b):
      rows, gsem, _ = bufs[b]
      pltpu.make_async_copy(g_s.at[src_v.at[0]], rows, gsem).wait()

    def scale(j, b):
      rows = bufs[b][0]
      rows[...] = rows[...] * ew_v[:, pl.ds(j, 1)]

    def s_start(j, b):
      rows, _, ssem = bufs[b]
      pltpu.async_copy(rows, acc.at[dst_v.at[j]], ssem, add=True)

    def s_wait(b):
      rows, _, ssem = bufs[b]
      pltpu.make_async_copy(rows, acc.at[dst_v.at[0]], ssem).wait()

    g_start(0, 0)
    g_start(1, 1)
    g_wait(0)
    scale(0, 0)
    s_start(0, 0)
    g_wait(1)
    scale(1, 1)
    s_wait(0)
    if rpw > 2:
      g_start(2, 0)
    s_start(1, 1)

    if rpw > 2:
      @pl.loop(2, rpw, step=2)
      def _(j):
        g_wait(0)
        scale(j, 0)
        s_wait(1)
        g_start(j + 1, 1)
        s_start(j, 0)
        g_wait(1)
        scale(j + 1, 1)
        s_wait(0)

        @pl.when(j + 2 < rpw)
        def _():
          g_start(j + 2, 0)

        s_start(j + 1, 1)

    s_wait(1)
    plsc.subcore_barrier()
    pltpu.sync_copy(acc.at[pl.ds(sid * rps, rps)],
                    out_hbm.at[cid, pl.ds(sid * rps, rps)])
    if tail:
      @pl.when(sid == 0)
      def _():
        pltpu.sync_copy(acc.at[pl.ds(rps * NS, tail)],
                        out_hbm.at[cid, pl.ds(rps * NS, tail)])

  return scat_kernel(src3d, dst3d, ewt3d, g, zeros_nf)


def _tc_encode(x, sv, f1w, f1b, f2w, f2b, w1a_t, w1b, deg_part):
  n = x.shape[0]
  h1 = w1a_t.shape[1]

  def body(x_ref, sv_ref, f1w_ref, f1b_ref, f2w_ref, f2b_ref, wa_ref, wb_ref,
           dp_ref, g1_ref, dinv_ref):
    sh = jnp.dot(sv_ref[...], f1w_ref[...], preferred_element_type=jnp.float32)
    sh = jnp.maximum(sh + f1b_ref[...], 0.0)
    senc = jnp.dot(sh, f2w_ref[...], preferred_element_type=jnp.float32)
    senc = senc + f2b_ref[...]
    hh = jnp.dot(x_ref[...], wa_ref[...], preferred_element_type=jnp.float32)
    hh = hh + jnp.dot(senc, wb_ref[...], preferred_element_type=jnp.float32)
    deg = dp_ref[0] + dp_ref[1] + 1.0
    dinv = lax.rsqrt(deg)
    g1_ref[...] = hh * dinv
    dinv_ref[...] = dinv

  return pl.pallas_call(
      body,
      out_shape=[
          jax.ShapeDtypeStruct((n, h1), jnp.float32),
          jax.ShapeDtypeStruct((n, 1), jnp.float32),
      ],
  )(x, sv, f1w, f1b, f2w, f2b, w1a_t, w1b, deg_part)


def _tc_mid(acc1, g1, dinv, b1, w2_t):
  n, h1 = g1.shape
  h2 = w2_t.shape[1]

  def body(a_ref, g_ref, d_ref, b_ref, w_ref, g2_ref):
    x1 = d_ref[...] * (a_ref[0] + a_ref[1] + g_ref[...]) + b_ref[...]
    act = jnp.where(x1 >= 0, x1, 0.01 * x1)
    hh = jnp.dot(act, w_ref[...], preferred_element_type=jnp.float32)
    g2_ref[...] = hh * d_ref[...]

  return pl.pallas_call(
      body,
      out_shape=jax.ShapeDtypeStruct((n, h2), jnp.float32),
  )(acc1, g1, dinv, b1, w2_t)


def _tc_final(acc2, g2, dinv, b2, fc_t, fc_b):
  n = g2.shape[0]

  def body(a_ref, g_ref, d_ref, b_ref, w_ref, fb_ref, out_ref):
    x2 = d_ref[...] * (a_ref[0] + a_ref[1] + g_ref[...]) + b_ref[...]
    act = jnp.where(x2 >= 0, x2, 0.01 * x2)
    out_ref[...] = (
        jnp.dot(act, w_ref[...], preferred_element_type=jnp.float32)
        + fb_ref[...]
    )

  return pl.pallas_call(
      body,
      out_shape=jax.ShapeDtypeStruct((n, 1), jnp.float32),
  )(acc2, g2, dinv, b2, fc_t, fc_b)


def kernel(x, edge_index, edge_weight, s_values, s_fc1_w, s_fc1_b, s_fc2_w,
           s_fc2_b, conv1_w, conv1_b, conv2_w, conv2_b, fc1_w, fc1_b):
  n, d = x.shape
  e = edge_weight.shape[0]
  h1 = conv1_w.shape[0]
  h2 = conv2_w.shape[0]

  nw = NC * NS
  rpw = e // (CH * nw)
  src3d = edge_index[0].reshape(nw, rpw, CH)
  dst3d = edge_index[1].reshape(nw, rpw, CH)
  ew3d = edge_weight.reshape(nw, rpw, CH)
  ewt3d = ew3d.transpose(0, 2, 1)
  zeros_n = jnp.zeros((n,), jnp.float32)
  zeros_nf1 = jnp.zeros((n, h1), jnp.float32)
  zeros_nf2 = jnp.zeros((n, h2), jnp.float32)

  deg_part = _sc_degree(dst3d, ew3d, zeros_n, n)
  deg_part3 = deg_part.reshape(NC, n, 1)

  g1, dinv = _tc_encode(
      x,
      s_values.reshape(1, -1),
      s_fc1_w.T,
      s_fc1_b.reshape(1, -1),
      s_fc2_w.T,
      s_fc2_b.reshape(1, -1),
      conv1_w[:, :d].T,
      conv1_w[:, d:].T,
      deg_part3,
  )

  acc1 = _sc_scatter(src3d, dst3d, ewt3d, g1, zeros_nf1, n, h1)
  g2 = _tc_mid(acc1, g1, dinv, conv1_b.reshape(1, -1), conv2_w.T)
  acc2 = _sc_scatter(src3d, dst3d, ewt3d, g2, zeros_nf2, n, h2)
  out = _tc_final(acc2, g2, dinv, conv2_b.reshape(1, -1), fc1_w.T,
                  fc1_b.reshape(1, -1))
  return out
